# baseline (speedup 1.0000x reference)
import functools

import jax
import jax.numpy as jnp
from jax import lax
from jax.experimental import pallas as pl
from jax.experimental.pallas import tpu as pltpu
from jax.experimental.pallas import tpu_sc as plsc

SEQ = 200
EMB = 64
NTAGS = 1000
LANES = 16
ROWS_PER_W = 8
ACTIVE_W = SEQ // ROWS_PER_W


def _sc_gather_pool(words, emb_table):
    mesh = plsc.VectorSubcoreMesh(core_axis_name="c", subcore_axis_name="s")

    @functools.partial(
        pl.kernel,
        out_type=jax.ShapeDtypeStruct((ACTIVE_W, EMB), jnp.float32),
        mesh=mesh,
        scratch_types=[
            pltpu.VMEM((ROWS_PER_W,), jnp.int32),
            pltpu.VMEM((ROWS_PER_W, EMB), jnp.float32),
            pltpu.VMEM((1, EMB), jnp.float32),
            pltpu.SemaphoreType.DMA,
        ],
        compiler_params=pltpu.CompilerParams(use_tc_tiling_on_sc=False),
    )
    def k(words_hbm, table_hbm, out_hbm, idx_v, rows_v, part_v, sem):
        wid = lax.axis_index("s") * 2 + lax.axis_index("c")

        @pl.when(wid < ACTIVE_W)
        def _():
            pltpu.sync_copy(
                words_hbm.at[pl.ds(wid * ROWS_PER_W, ROWS_PER_W)], idx_v)
            pltpu.async_copy(table_hbm.at[idx_v], rows_v, sem).wait()
            for c in range(EMB // LANES):
                acc = rows_v[0, pl.ds(c * LANES, LANES)]
                for j in range(1, ROWS_PER_W):
                    acc = acc + rows_v[j, pl.ds(c * LANES, LANES)]
                part_v[0, pl.ds(c * LANES, LANES)] = acc
            pltpu.sync_copy(part_v, out_hbm.at[pl.ds(wid, 1)])

    return k(words, emb_table)


def _tc_mlp(partials, W0, b0, W1, b1, W_out, b_out):
    def body(p_ref, w0_ref, b0_ref, w1_ref, b1_ref, wo_ref, bo_ref, o_ref):
        h = jnp.sum(p_ref[...], axis=0, keepdims=True)
        h = jnp.tanh(
            lax.dot_general(h, w0_ref[...], (((1,), (1,)), ((), ())),
                            preferred_element_type=jnp.float32) + b0_ref[...])
        h = jnp.tanh(
            lax.dot_general(h, w1_ref[...], (((1,), (1,)), ((), ())),
                            preferred_element_type=jnp.float32) + b1_ref[...])
        o_ref[...] = lax.dot_general(
            h, wo_ref[...], (((1,), (1,)), ((), ())),
            preferred_element_type=jnp.float32) + bo_ref[...]

    return pl.pallas_call(
        body,
        out_shape=jax.ShapeDtypeStruct((1, NTAGS), jnp.float32),
    )(partials, W0, b0, W1, b1, W_out, b_out)


def kernel(words, emb_table, W0, b0, W1, b1, W_out, b_out):
    partials = _sc_gather_pool(words.astype(jnp.int32), emb_table)
    return _tc_mlp(partials, W0, b0.reshape(1, -1), W1, b1.reshape(1, -1),
                   W_out, b_out.reshape(1, -1))


# speedup vs baseline: 20.2044x; 20.2044x over previous
import functools

import jax
import jax.numpy as jnp
from jax import lax
from jax.experimental import pallas as pl
from jax.experimental.pallas import tpu as pltpu
from jax.experimental.pallas import tpu_sc as plsc

SEQ = 200
EMB = 64
NTAGS = 1000
LANES = 16
ROWS_PER_W = 8
ACTIVE_W = SEQ // ROWS_PER_W
BLK = 128


def _sc_gather_pool(words, emb_table_t):
    mesh = plsc.VectorSubcoreMesh(core_axis_name="c", subcore_axis_name="s")

    @functools.partial(
        pl.kernel,
        out_type=jax.ShapeDtypeStruct((ACTIVE_W, EMB), jnp.float32),
        mesh=mesh,
        scratch_types=[
            pltpu.VMEM((LANES,), jnp.int32),
            pltpu.VMEM((ROWS_PER_W * EMB + 1, BLK), jnp.float32),
            pltpu.VMEM((1, EMB), jnp.float32),
            pltpu.SemaphoreType.DMA,
        ],
    )
    def k(words_hbm, table_hbm, out_hbm, idx_v, blocks_v, part_v, sem):
        wid = lax.axis_index("s") * 2 + lax.axis_index("c")

        @pl.when(wid < ACTIVE_W)
        def _():
            pltpu.sync_copy(
                words_hbm.at[pl.ds(wid * ROWS_PER_W, ROWS_PER_W)],
                idx_v.at[pl.ds(0, ROWS_PER_W)])
            idx_vec = idx_v[...]
            for j in range(ROWS_PER_W):
                tb = pl.multiple_of((idx_vec[j] // BLK) * BLK, BLK)
                pltpu.async_copy(
                    table_hbm.at[:, pl.ds(tb, BLK)],
                    blocks_v.at[pl.ds(j * EMB, EMB)], sem)
            for j in range(ROWS_PER_W):
                pltpu.make_async_copy(
                    table_hbm.at[:, pl.ds(0, BLK)],
                    blocks_v.at[pl.ds(j * EMB, EMB)], sem).wait()
            acc = [jnp.float32(0.0)] * EMB
            for j in range(ROWS_PER_W):
                col = idx_vec[j] % BLK
                for d in range(EMB):
                    v = blocks_v[j * EMB + d, pl.ds(col, LANES)]
                    acc[d] = acc[d] + v[0]
            lane = lax.iota(jnp.int32, LANES)
            ones = jnp.full((LANES,), 1.0, jnp.float32)
            for c in range(EMB // LANES):
                vec = jnp.zeros((LANES,), jnp.float32)
                for l in range(LANES):
                    vec = jnp.where(lane == l, ones * acc[c * LANES + l], vec)
                part_v[0, pl.ds(c * LANES, LANES)] = vec
            pltpu.sync_copy(part_v, out_hbm.at[pl.ds(wid, 1)])

    return k(words, emb_table_t)


def _tc_mlp(partials, W0, b0, W1, b1, W_out, b_out):
    def body(p_ref, w0_ref, b0_ref, w1_ref, b1_ref, wo_ref, bo_ref, o_ref):
        h = jnp.sum(p_ref[...], axis=0, keepdims=True)
        h = jnp.tanh(
            lax.dot_general(h, w0_ref[...], (((1,), (1,)), ((), ())),
                            preferred_element_type=jnp.float32) + b0_ref[...])
        h = jnp.tanh(
            lax.dot_general(h, w1_ref[...], (((1,), (1,)), ((), ())),
                            preferred_element_type=jnp.float32) + b1_ref[...])
        o_ref[...] = lax.dot_general(
            h, wo_ref[...], (((1,), (1,)), ((), ())),
            preferred_element_type=jnp.float32) + bo_ref[...]

    return pl.pallas_call(
        body,
        out_shape=jax.ShapeDtypeStruct((1, NTAGS), jnp.float32),
    )(partials, W0, b0, W1, b1, W_out, b_out)


def kernel(words, emb_table, W0, b0, W1, b1, W_out, b_out):
    partials = _sc_gather_pool(words.astype(jnp.int32), emb_table.T)
    return _tc_mlp(partials, W0, b0.reshape(1, -1), W1, b1.reshape(1, -1),
                   W_out, b_out.reshape(1, -1))


# speedup vs baseline: 22.4424x; 1.1108x over previous
import functools

import jax
import jax.numpy as jnp
from jax import lax
from jax.experimental import pallas as pl
from jax.experimental.pallas import tpu as pltpu
from jax.experimental.pallas import tpu_sc as plsc

SEQ = 200
EMB = 64
NTAGS = 1000
LANES = 16
ROWS_PER_W = 8
ACTIVE_W = SEQ // ROWS_PER_W
BLK = 128


def _sc_gather_pool(words, emb_table_t):
    mesh = plsc.VectorSubcoreMesh(core_axis_name="c", subcore_axis_name="s")

    @functools.partial(
        pl.kernel,
        out_type=jax.ShapeDtypeStruct((ACTIVE_W, EMB), jnp.float32),
        mesh=mesh,
        scratch_types=[
            pltpu.VMEM((LANES,), jnp.int32),
            pltpu.VMEM((ROWS_PER_W * EMB + 2, BLK), jnp.float32),
            pltpu.VMEM((1, EMB), jnp.float32),
            pltpu.SemaphoreType.DMA,
        ],
    )
    def k(words_hbm, table_hbm, out_hbm, idx_v, blocks_v, part_v, sem):
        wid = lax.axis_index("s") * 2 + lax.axis_index("c")

        @pl.when(wid < ACTIVE_W)
        def _():
            pltpu.sync_copy(
                words_hbm.at[pl.ds(wid * ROWS_PER_W, ROWS_PER_W)],
                idx_v.at[pl.ds(0, ROWS_PER_W)])
            idx_vec = idx_v[...]
            for j in range(ROWS_PER_W):
                tb = pl.multiple_of((idx_vec[j] // BLK) * BLK, BLK)
                pltpu.async_copy(
                    table_hbm.at[:, pl.ds(tb, BLK)],
                    blocks_v.at[pl.ds(1 + j * EMB, EMB)], sem)
            for j in range(ROWS_PER_W):
                pltpu.make_async_copy(
                    table_hbm.at[:, pl.ds(0, BLK)],
                    blocks_v.at[pl.ds(1 + j * EMB, EMB)], sem).wait()
            lane = lax.iota(jnp.int32, LANES)
            acc = [jnp.zeros((LANES,), jnp.float32)
                   for _ in range(EMB // LANES)]
            for j in range(ROWS_PER_W):
                col = idx_vec[j] % BLK
                for c in range(EMB // LANES):
                    for l in range(LANES):
                        d = c * LANES + l
                        v = blocks_v[1 + j * EMB + d, pl.ds(col - l, LANES)]
                        acc[c] = jnp.where(lane == l, acc[c] + v, acc[c])
            for c in range(EMB // LANES):
                part_v[0, pl.ds(c * LANES, LANES)] = acc[c]
            pltpu.sync_copy(part_v, out_hbm.at[pl.ds(wid, 1)])

    return k(words, emb_table_t)


def _tc_mlp(partials, W0, b0, W1, b1, W_out, b_out):
    def body(p_ref, w0_ref, b0_ref, w1_ref, b1_ref, wo_ref, bo_ref, o_ref):
        h = jnp.sum(p_ref[...], axis=0, keepdims=True)
        h = jnp.tanh(
            lax.dot_general(h, w0_ref[...], (((1,), (1,)), ((), ())),
                            preferred_element_type=jnp.float32) + b0_ref[...])
        h = jnp.tanh(
            lax.dot_general(h, w1_ref[...], (((1,), (1,)), ((), ())),
                            preferred_element_type=jnp.float32) + b1_ref[...])
        o_ref[...] = lax.dot_general(
            h, wo_ref[...], (((1,), (1,)), ((), ())),
            preferred_element_type=jnp.float32) + bo_ref[...]

    return pl.pallas_call(
        body,
        out_shape=jax.ShapeDtypeStruct((1, NTAGS), jnp.float32),
    )(partials, W0, b0, W1, b1, W_out, b_out)


def kernel(words, emb_table, W0, b0, W1, b1, W_out, b_out):
    partials = _sc_gather_pool(words.astype(jnp.int32), emb_table.T)
    return _tc_mlp(partials, W0, b0.reshape(1, -1), W1, b1.reshape(1, -1),
                   W_out, b_out.reshape(1, -1))


# speedup vs baseline: 24.0523x; 1.0717x over previous
import functools

import jax
import jax.numpy as jnp
from jax import lax
from jax.experimental import pallas as pl
from jax.experimental.pallas import tpu as pltpu
from jax.experimental.pallas import tpu_sc as plsc

SEQ = 200
EMB = 64
NTAGS = 1000
LANES = 16
ROWS_PER_W = 8
ACTIVE_W = SEQ // ROWS_PER_W
BLK = 128


def _sc_gather_pool(words, emb_table_t):
    mesh = plsc.VectorSubcoreMesh(core_axis_name="c", subcore_axis_name="s")

    @functools.partial(
        pl.kernel,
        out_type=jax.ShapeDtypeStruct((ACTIVE_W, EMB), jnp.float32),
        mesh=mesh,
        scratch_types=[
            pltpu.VMEM((ROWS_PER_W + LANES,), jnp.int32),
            pltpu.VMEM((ROWS_PER_W * EMB + 2, BLK), jnp.float32),
            pltpu.VMEM((1, EMB), jnp.float32),
            pltpu.SemaphoreType.DMA,
        ],
    )
    def k(words_hbm, table_hbm, out_hbm, idx_v, blocks_v, part_v, sem):
        wid = lax.axis_index("s") * 2 + lax.axis_index("c")

        @pl.when(wid < ACTIVE_W)
        def _():
            pltpu.sync_copy(
                words_hbm.at[pl.ds(wid * ROWS_PER_W, ROWS_PER_W)],
                idx_v.at[pl.ds(0, ROWS_PER_W)])
            idx_vec = idx_v[pl.ds(0, LANES)]
            for j in range(ROWS_PER_W):
                tb = pl.multiple_of((idx_vec[j] // BLK) * BLK, BLK)
                pltpu.async_copy(
                    table_hbm.at[:, pl.ds(tb, BLK)],
                    blocks_v.at[pl.ds(1 + j * EMB, EMB)], sem)
            for j in range(ROWS_PER_W):
                pltpu.make_async_copy(
                    table_hbm.at[:, pl.ds(0, BLK)],
                    blocks_v.at[pl.ds(1 + j * EMB, EMB)], sem).wait()
            lane = lax.iota(jnp.int32, LANES)

            def word_body(j, acc):
                col = idx_v[pl.ds(j, LANES)][0] % BLK
                for c in range(EMB // LANES):
                    for l in range(LANES):
                        d = c * LANES + l
                        v = blocks_v[1 + j * EMB + d, pl.ds(col - l, LANES)]
                        acc = (acc[:c]
                               + (jnp.where(lane == l, acc[c] + v, acc[c]),)
                               + acc[c + 1:])
                return acc

            acc = lax.fori_loop(
                0, ROWS_PER_W, word_body,
                tuple(jnp.zeros((LANES,), jnp.float32)
                      for _ in range(EMB // LANES)))
            for c in range(EMB // LANES):
                part_v[0, pl.ds(c * LANES, LANES)] = acc[c]
            pltpu.sync_copy(part_v, out_hbm.at[pl.ds(wid, 1)])

    return k(words, emb_table_t)


def _tc_mlp(partials, W0, b0, W1, b1, W_out, b_out):
    def body(p_ref, w0_ref, b0_ref, w1_ref, b1_ref, wo_ref, bo_ref, o_ref):
        h = jnp.sum(p_ref[...], axis=0, keepdims=True)
        h = jnp.tanh(
            lax.dot_general(h, w0_ref[...], (((1,), (1,)), ((), ())),
                            preferred_element_type=jnp.float32) + b0_ref[...])
        h = jnp.tanh(
            lax.dot_general(h, w1_ref[...], (((1,), (1,)), ((), ())),
                            preferred_element_type=jnp.float32) + b1_ref[...])
        o_ref[...] = lax.dot_general(
            h, wo_ref[...], (((1,), (1,)), ((), ())),
            preferred_element_type=jnp.float32) + bo_ref[...]

    return pl.pallas_call(
        body,
        out_shape=jax.ShapeDtypeStruct((1, NTAGS), jnp.float32),
    )(partials, W0, b0, W1, b1, W_out, b_out)


def kernel(words, emb_table, W0, b0, W1, b1, W_out, b_out):
    partials = _sc_gather_pool(words.astype(jnp.int32), emb_table.T)
    return _tc_mlp(partials, W0, b0.reshape(1, -1), W1, b1.reshape(1, -1),
                   W_out, b_out.reshape(1, -1))
